# baseline (speedup 1.0000x reference)
import functools

import jax
import jax.numpy as jnp
from jax import lax
from jax.experimental import pallas as pl
from jax.experimental.pallas import tpu as pltpu
from jax.experimental.pallas import tpu_sc as plsc

_NUM_CHARS = 100000
_HIDDEN = 16
_BATCH = 1024
_TM = 1000
_NSTEP = _NUM_CHARS // _TM
_NBUF = 4


@functools.cache
def _make_sc_gather():
    info = plsc.get_sparse_core_info()
    nc, ns = info.num_cores, info.num_subcores
    nw = nc * ns
    b_per_w = _BATCH // nw
    mesh = plsc.VectorSubcoreMesh(core_axis_name="c", subcore_axis_name="s")

    @functools.partial(
        pl.kernel,
        mesh=mesh,
        out_type=jax.ShapeDtypeStruct((_BATCH, _HIDDEN), jnp.float32),
        scratch_types=[
            pltpu.VMEM((b_per_w,), jnp.int32),
            pltpu.VMEM((b_per_w, _HIDDEN), jnp.float32),
            pltpu.SemaphoreType.DMA,
        ],
        compiler_params=pltpu.CompilerParams(use_tc_tiling_on_sc=False),
    )
    def gather_k(idx_hbm, table_hbm, out_hbm, idx_v, rows_v, sem):
        wid = lax.axis_index("s") * nc + lax.axis_index("c")
        base = wid * b_per_w
        pltpu.sync_copy(idx_hbm.at[pl.ds(base, b_per_w)], idx_v)
        pltpu.async_copy(table_hbm.at[idx_v], rows_v, sem).wait()
        pltpu.sync_copy(rows_v, out_hbm.at[pl.ds(base, b_per_w)])

    return gather_k


def _out_copy(scratch, out_hbm, sems, step, buf):
    return pltpu.make_async_copy(
        scratch.at[buf],
        out_hbm.at[pl.ds(step * _TM, _TM)],
        sems.at[buf],
    )


def _matmul_body(x_ref, proj_ref, out_hbm, scratch, sems):
    i = pl.program_id(0)
    j = lax.rem(i, _NBUF)

    @pl.when(i >= _NBUF)
    def _():
        _out_copy(scratch, out_hbm, sems, i - _NBUF, j).wait()

    scratch[j] = lax.dot_general(
        proj_ref[...],
        x_ref[...],
        dimension_numbers=(((1,), (1,)), ((), ())),
        preferred_element_type=jnp.float32,
    )
    _out_copy(scratch, out_hbm, sems, i, j).start()

    @pl.when(i == _NSTEP - 1)
    def _():
        for k in range(_NBUF):
            step = _NSTEP - _NBUF + k
            _out_copy(scratch, out_hbm, sems, step, step % _NBUF).wait()


def _tc_matmul(x, proj):
    return pl.pallas_call(
        _matmul_body,
        grid=(_NSTEP,),
        in_specs=[
            pl.BlockSpec((_BATCH, _HIDDEN), lambda i: (0, 0)),
            pl.BlockSpec((_TM, _HIDDEN), lambda i: (i, 0)),
        ],
        out_specs=pl.BlockSpec(memory_space=pl.ANY),
        out_shape=jax.ShapeDtypeStruct((_NUM_CHARS, _BATCH), jnp.float32),
        scratch_shapes=[
            pltpu.VMEM((_NBUF, _TM, _BATCH), jnp.float32),
            pltpu.SemaphoreType.DMA((_NBUF,)),
        ],
    )(x, proj)


def kernel(indices, embedding, projection_matrix):
    x = _make_sc_gather()(indices.astype(jnp.int32), embedding)
    return _tc_matmul(x, projection_matrix)


# speedup vs baseline: 1.0327x; 1.0327x over previous
import functools

import jax
import jax.numpy as jnp
from jax import lax
from jax.experimental import pallas as pl
from jax.experimental.pallas import tpu as pltpu
from jax.experimental.pallas import tpu_sc as plsc

_NUM_CHARS = 100000
_HIDDEN = 16
_BATCH = 1024
_TM = 5000


@functools.cache
def _make_sc_gather():
    info = plsc.get_sparse_core_info()
    nc, ns = info.num_cores, info.num_subcores
    nw = nc * ns
    b_per_w = _BATCH // nw
    mesh = plsc.VectorSubcoreMesh(core_axis_name="c", subcore_axis_name="s")

    @functools.partial(
        pl.kernel,
        mesh=mesh,
        out_type=jax.ShapeDtypeStruct((_BATCH, _HIDDEN), jnp.float32),
        scratch_types=[
            pltpu.VMEM((b_per_w,), jnp.int32),
            pltpu.VMEM((b_per_w, _HIDDEN), jnp.float32),
            pltpu.SemaphoreType.DMA,
        ],
        compiler_params=pltpu.CompilerParams(use_tc_tiling_on_sc=False),
    )
    def gather_k(idx_hbm, table_hbm, out_hbm, idx_v, rows_v, sem):
        wid = lax.axis_index("s") * nc + lax.axis_index("c")
        base = wid * b_per_w
        pltpu.sync_copy(idx_hbm.at[pl.ds(base, b_per_w)], idx_v)
        pltpu.async_copy(table_hbm.at[idx_v], rows_v, sem).wait()
        pltpu.sync_copy(rows_v, out_hbm.at[pl.ds(base, b_per_w)])

    return gather_k


def _matmul_body(x_ref, proj_ref, out_ref):
    out_ref[...] = lax.dot_general(
        proj_ref[...],
        x_ref[...],
        dimension_numbers=(((1,), (1,)), ((), ())),
        preferred_element_type=jnp.float32,
    )


def _tc_matmul(x, proj):
    return pl.pallas_call(
        _matmul_body,
        grid=(_NUM_CHARS // _TM,),
        in_specs=[
            pl.BlockSpec((_BATCH, _HIDDEN), lambda i: (0, 0)),
            pl.BlockSpec((_TM, _HIDDEN), lambda i: (i, 0)),
        ],
        out_specs=pl.BlockSpec((_TM, _BATCH), lambda i: (i, 0)),
        out_shape=jax.ShapeDtypeStruct((_NUM_CHARS, _BATCH), jnp.float32),
    )(x, proj)


def kernel(indices, embedding, projection_matrix):
    x = _make_sc_gather()(indices.astype(jnp.int32), embedding)
    return _tc_matmul(x, projection_matrix)


# speedup vs baseline: 1.0404x; 1.0075x over previous
import functools

import jax
import jax.numpy as jnp
from jax import lax
from jax.experimental import pallas as pl
from jax.experimental.pallas import tpu as pltpu
from jax.experimental.pallas import tpu_sc as plsc

_NUM_CHARS = 100000
_HIDDEN = 16
_BATCH = 1024
_TM = 5000
_SUPER = 128 // _HIDDEN


@functools.cache
def _make_sc_gather():
    info = plsc.get_sparse_core_info()
    nc, ns, nl = info.num_cores, info.num_subcores, info.num_lanes
    nw = nc * ns
    b_per_w = _BATCH // nw
    mesh = plsc.VectorSubcoreMesh(core_axis_name="c", subcore_axis_name="s")

    @functools.partial(
        pl.kernel,
        mesh=mesh,
        out_type=jax.ShapeDtypeStruct((_BATCH, _HIDDEN), jnp.float32),
        scratch_types=[
            pltpu.VMEM((b_per_w,), jnp.int32),
            pltpu.VMEM((b_per_w,), jnp.int32),
            pltpu.VMEM((b_per_w, 128), jnp.float32),
            pltpu.VMEM((b_per_w, _HIDDEN), jnp.float32),
            pltpu.SemaphoreType.DMA,
        ],
        compiler_params=pltpu.CompilerParams(needs_layout_passes=False),
    )
    def gather_k(idx_hbm, table_hbm, out_hbm, idx_v, sidx_v, rows_v, x_v, sem):
        wid = lax.axis_index("s") * nc + lax.axis_index("c")
        base = wid * b_per_w
        pltpu.sync_copy(idx_hbm.at[pl.ds(base, b_per_w)], idx_v)
        for c in range(b_per_w // nl):
            v = idx_v[pl.ds(c * nl, nl)]
            sidx_v[pl.ds(c * nl, nl)] = lax.shift_right_logical(v, 3)
        pltpu.async_copy(table_hbm.at[sidx_v], rows_v, sem).wait()
        iota = lax.iota(jnp.int32, nl)
        for c in range(b_per_w // nl):
            v = idx_v[pl.ds(c * nl, nl)]
            col_base = lax.bitwise_and(v, 7) * _HIDDEN
            row_ids = iota + c * nl
            for l in range(_HIDDEN):
                vals = plsc.load_gather(rows_v, [row_ids, col_base + l])
                plsc.store_scatter(
                    x_v, [row_ids, jnp.full((nl,), l, jnp.int32)], vals)
        pltpu.sync_copy(x_v, out_hbm.at[pl.ds(base, b_per_w)])

    return gather_k


def _matmul_body(x_ref, proj_ref, out_ref):
    out_ref[...] = lax.dot_general(
        proj_ref[...],
        x_ref[...],
        dimension_numbers=(((1,), (1,)), ((), ())),
        preferred_element_type=jnp.float32,
    )


def _tc_matmul(x, proj):
    return pl.pallas_call(
        _matmul_body,
        grid=(_NUM_CHARS // _TM,),
        in_specs=[
            pl.BlockSpec((_BATCH, _HIDDEN), lambda i: (0, 0)),
            pl.BlockSpec((_TM, _HIDDEN), lambda i: (i, 0)),
        ],
        out_specs=pl.BlockSpec((_TM, _BATCH), lambda i: (i, 0)),
        out_shape=jax.ShapeDtypeStruct((_NUM_CHARS, _BATCH), jnp.float32),
    )(x, proj)


def kernel(indices, embedding, projection_matrix):
    table = embedding.reshape(_NUM_CHARS // _SUPER, 8 * _HIDDEN)
    x = _make_sc_gather()(indices.astype(jnp.int32), table)
    return _tc_matmul(x, projection_matrix)


# speedup vs baseline: 1.2279x; 1.1802x over previous
import functools

import jax
import jax.numpy as jnp
from jax import lax
from jax.experimental import pallas as pl
from jax.experimental.pallas import tpu as pltpu
from jax.experimental.pallas import tpu_sc as plsc

_NUM_CHARS = 100000
_HIDDEN = 16
_BATCH = 1024
_TM = 4096


@functools.cache
def _make_sc_gather():
    info = plsc.get_sparse_core_info()
    nc, ns = info.num_cores, info.num_subcores
    nw = nc * ns
    b_per_w = _BATCH // nw
    mesh = plsc.VectorSubcoreMesh(core_axis_name="c", subcore_axis_name="s")

    @functools.partial(
        pl.kernel,
        mesh=mesh,
        out_type=jax.ShapeDtypeStruct((_BATCH, _HIDDEN), jnp.float32),
        scratch_types=[
            pltpu.VMEM((b_per_w,), jnp.int32),
            pltpu.VMEM((b_per_w, _HIDDEN), jnp.float32),
            pltpu.SemaphoreType.DMA,
        ],
        compiler_params=pltpu.CompilerParams(use_tc_tiling_on_sc=False),
    )
    def gather_k(idx_hbm, table_hbm, out_hbm, idx_v, rows_v, sem):
        wid = lax.axis_index("s") * nc + lax.axis_index("c")
        base = wid * b_per_w
        pltpu.sync_copy(idx_hbm.at[pl.ds(base, b_per_w)], idx_v)
        pltpu.async_copy(table_hbm.at[idx_v], rows_v, sem).wait()
        pltpu.sync_copy(rows_v, out_hbm.at[pl.ds(base, b_per_w)])

    return gather_k


def _matmul_body(x_ref, projt_ref, out_ref):
    out_ref[...] = lax.dot_general(
        projt_ref[...],
        x_ref[...],
        dimension_numbers=(((0,), (1,)), ((), ())),
        preferred_element_type=jnp.float32,
    )


def _tc_matmul(x, projt):
    return pl.pallas_call(
        _matmul_body,
        grid=(pl.cdiv(_NUM_CHARS, _TM),),
        in_specs=[
            pl.BlockSpec((_BATCH, _HIDDEN), lambda i: (0, 0)),
            pl.BlockSpec((_HIDDEN, _TM), lambda i: (0, i)),
        ],
        out_specs=pl.BlockSpec((_TM, _BATCH), lambda i: (i, 0)),
        out_shape=jax.ShapeDtypeStruct((_NUM_CHARS, _BATCH), jnp.float32),
    )(x, projt)


def kernel(indices, embedding, projection_matrix):
    x = _make_sc_gather()(indices.astype(jnp.int32), embedding)
    return _tc_matmul(x, projection_matrix.T)


# speedup vs baseline: 1.5116x; 1.2310x over previous
import functools

import jax
import jax.numpy as jnp
from jax import lax
from jax.experimental import pallas as pl
from jax.experimental.pallas import tpu as pltpu
from jax.experimental.pallas import tpu_sc as plsc

_NUM_CHARS = 100000
_HIDDEN = 16
_BATCH = 1024
_TM = 4096
_LANE_TILES = 782
_SLAB = 3200


@functools.cache
def _make_sc_gather():
    info = plsc.get_sparse_core_info()
    nc, ns, nl = info.num_cores, info.num_subcores, info.num_lanes
    nw = nc * ns
    mesh = plsc.VectorSubcoreMesh(core_axis_name="c", subcore_axis_name="s")

    @functools.partial(
        pl.kernel,
        mesh=mesh,
        out_type=jax.ShapeDtypeStruct((nw, _HIDDEN, _BATCH), jnp.float32),
        scratch_types=[
            pltpu.VMEM((_BATCH,), jnp.int32),
            pltpu.VMEM((_HIDDEN, _SLAB), jnp.float32),
            pltpu.VMEM((_HIDDEN, _BATCH), jnp.float32),
        ],
        compiler_params=pltpu.CompilerParams(needs_layout_passes=False),
    )
    def gather_k(idx_hbm, tablet_hbm, out_hbm, idx_v, slab_v, xt_v):
        wid = lax.axis_index("s") * nc + lax.axis_index("c")
        lo_tile = (_LANE_TILES * wid) // nw
        hi_tile = (_LANE_TILES * (wid + 1)) // nw
        own_lo = lo_tile * 128
        own_hi = hi_tile * 128
        pltpu.sync_copy(idx_hbm, idx_v)
        pltpu.sync_copy(
            tablet_hbm.at[:, pl.ds(pl.multiple_of(own_lo, 128), _SLAB)],
            slab_v,
        )
        iota = lax.iota(jnp.int32, nl)

        def chunk(c, carry):
            v = idx_v[pl.ds(c * nl, nl)]
            m = (v >= own_lo) & (v < own_hi)
            loc = jnp.clip(v - own_lo, 0, _SLAB - 1)
            cols = c * nl + iota
            for h in range(_HIDDEN):
                g = plsc.load_gather(
                    slab_v, [jnp.full((nl,), h, jnp.int32), loc])
                z = jnp.where(m, g, jnp.float32(0.0))
                plsc.store_scatter(
                    xt_v, [jnp.full((nl,), h, jnp.int32), cols], z)
            return carry

        lax.fori_loop(0, _BATCH // nl, chunk, 0, unroll=False)
        pltpu.sync_copy(xt_v, out_hbm.at[wid])

    return gather_k


def _matmul_body(xs_ref, projt_ref, out_ref, xt_vmem):
    i = pl.program_id(0)

    @pl.when(i == 0)
    def _():
        xt_vmem[...] = jnp.sum(xs_ref[...], axis=0)

    out_ref[...] = lax.dot_general(
        projt_ref[...],
        xt_vmem[...],
        dimension_numbers=(((0,), (0,)), ((), ())),
        preferred_element_type=jnp.float32,
    )


def _tc_matmul(xs, projt):
    nw = xs.shape[0]
    return pl.pallas_call(
        _matmul_body,
        grid=(pl.cdiv(_NUM_CHARS, _TM),),
        in_specs=[
            pl.BlockSpec((nw, _HIDDEN, _BATCH), lambda i: (0, 0, 0)),
            pl.BlockSpec((_HIDDEN, _TM), lambda i: (0, i)),
        ],
        out_specs=pl.BlockSpec((_TM, _BATCH), lambda i: (i, 0)),
        out_shape=jax.ShapeDtypeStruct((_NUM_CHARS, _BATCH), jnp.float32),
        scratch_shapes=[pltpu.VMEM((_HIDDEN, _BATCH), jnp.float32)],
    )(xs, projt)


def kernel(indices, embedding, projection_matrix):
    xs = _make_sc_gather()(indices.astype(jnp.int32), embedding.T)
    return _tc_matmul(xs, projection_matrix.T)


# speedup vs baseline: 1.5169x; 1.0035x over previous
import functools

import jax
import jax.numpy as jnp
from jax import lax
from jax.experimental import pallas as pl
from jax.experimental.pallas import tpu as pltpu
from jax.experimental.pallas import tpu_sc as plsc

_NUM_CHARS = 100000
_HIDDEN = 16
_BATCH = 1024
_TM = 4096
_LANE_TILES = 782
_SLAB = 3200


@functools.cache
def _make_sc_gather():
    info = plsc.get_sparse_core_info()
    nc, ns, nl = info.num_cores, info.num_subcores, info.num_lanes
    nw = nc * ns
    mesh = plsc.VectorSubcoreMesh(core_axis_name="c", subcore_axis_name="s")

    @functools.partial(
        pl.kernel,
        mesh=mesh,
        out_type=jax.ShapeDtypeStruct((nw, _HIDDEN, _BATCH), jnp.float32),
        scratch_types=[
            pltpu.VMEM((_BATCH,), jnp.int32),
            pltpu.VMEM((_HIDDEN, _SLAB), jnp.float32),
            pltpu.VMEM((_HIDDEN, _BATCH), jnp.float32),
        ],
        compiler_params=pltpu.CompilerParams(needs_layout_passes=False),
    )
    def gather_k(idx_hbm, tablet_hbm, out_hbm, idx_v, slab_v, xt_v):
        wid = lax.axis_index("s") * nc + lax.axis_index("c")
        lo_tile = (_LANE_TILES * wid) // nw
        hi_tile = (_LANE_TILES * (wid + 1)) // nw
        own_lo = lo_tile * 128
        own_hi = hi_tile * 128
        pltpu.sync_copy(idx_hbm, idx_v)
        pltpu.sync_copy(
            tablet_hbm.at[:, pl.ds(pl.multiple_of(own_lo, 128), _SLAB)],
            slab_v,
        )
        iota = lax.iota(jnp.int32, nl)

        zeros = jnp.zeros((nl,), jnp.float32)

        def chunk(c, carry):
            v = idx_v[pl.ds(c * nl, nl)]
            m = (v >= own_lo) & (v < own_hi)
            loc = jnp.clip(v - own_lo, 0, _SLAB - 1)
            cols = c * nl + iota

            def hit():
                for h in range(_HIDDEN):
                    hv = jnp.full((nl,), h, jnp.int32)
                    g = plsc.load_gather(slab_v, [hv, loc])
                    plsc.store_scatter(
                        xt_v, [hv, cols], jnp.where(m, g, jnp.float32(0.0)))

            def miss():
                for h in range(_HIDDEN):
                    hv = jnp.full((nl,), h, jnp.int32)
                    plsc.store_scatter(xt_v, [hv, cols], zeros)

            lax.cond(jnp.any(m), hit, miss)
            return carry

        lax.fori_loop(0, _BATCH // nl, chunk, 0, unroll=False)
        pltpu.sync_copy(xt_v, out_hbm.at[wid])

    return gather_k


def _matmul_body(xs_ref, projt_ref, out_ref, xt_vmem):
    i = pl.program_id(0)

    @pl.when(i == 0)
    def _():
        xt_vmem[...] = jnp.sum(xs_ref[...], axis=0)

    out_ref[...] = lax.dot_general(
        projt_ref[...],
        xt_vmem[...],
        dimension_numbers=(((0,), (0,)), ((), ())),
        preferred_element_type=jnp.float32,
    )


def _tc_matmul(xs, projt):
    nw = xs.shape[0]
    return pl.pallas_call(
        _matmul_body,
        grid=(pl.cdiv(_NUM_CHARS, _TM),),
        in_specs=[
            pl.BlockSpec((nw, _HIDDEN, _BATCH), lambda i: (0, 0, 0)),
            pl.BlockSpec((_HIDDEN, _TM), lambda i: (0, i)),
        ],
        out_specs=pl.BlockSpec((_TM, _BATCH), lambda i: (i, 0)),
        out_shape=jax.ShapeDtypeStruct((_NUM_CHARS, _BATCH), jnp.float32),
        scratch_shapes=[pltpu.VMEM((_HIDDEN, _BATCH), jnp.float32)],
    )(xs, projt)


def kernel(indices, embedding, projection_matrix):
    xs = _make_sc_gather()(indices.astype(jnp.int32), embedding.T)
    return _tc_matmul(xs, projection_matrix.T)


# speedup vs baseline: 1.5272x; 1.0068x over previous
import functools

import jax
import jax.numpy as jnp
from jax import lax
from jax.experimental import pallas as pl
from jax.experimental.pallas import tpu as pltpu
from jax.experimental.pallas import tpu_sc as plsc

_NUM_CHARS = 100000
_HIDDEN = 16
_BATCH = 1024
_TM = 2048
_LANE_TILES = 782
_SLAB = 3200


@functools.cache
def _make_sc_gather():
    info = plsc.get_sparse_core_info()
    nc, ns, nl = info.num_cores, info.num_subcores, info.num_lanes
    nw = nc * ns
    mesh = plsc.VectorSubcoreMesh(core_axis_name="c", subcore_axis_name="s")

    @functools.partial(
        pl.kernel,
        mesh=mesh,
        out_type=jax.ShapeDtypeStruct((nw, _HIDDEN, _BATCH), jnp.float32),
        scratch_types=[
            pltpu.VMEM((_BATCH,), jnp.int32),
            pltpu.VMEM((_HIDDEN, _SLAB), jnp.float32),
            pltpu.VMEM((_HIDDEN, _BATCH), jnp.float32),
        ],
        compiler_params=pltpu.CompilerParams(needs_layout_passes=False),
    )
    def gather_k(idx_hbm, tablet_hbm, out_hbm, idx_v, slab_v, xt_v):
        wid = lax.axis_index("s") * nc + lax.axis_index("c")
        lo_tile = (_LANE_TILES * wid) // nw
        hi_tile = (_LANE_TILES * (wid + 1)) // nw
        own_lo = lo_tile * 128
        own_hi = hi_tile * 128
        pltpu.sync_copy(idx_hbm, idx_v)
        pltpu.sync_copy(
            tablet_hbm.at[:, pl.ds(pl.multiple_of(own_lo, 128), _SLAB)],
            slab_v,
        )
        iota = lax.iota(jnp.int32, nl)

        zeros = jnp.zeros((nl,), jnp.float32)

        def chunk(c, carry):
            v = idx_v[pl.ds(c * nl, nl)]
            m = (v >= own_lo) & (v < own_hi)
            loc = jnp.clip(v - own_lo, 0, _SLAB - 1)
            cols = c * nl + iota

            def hit():
                for h in range(_HIDDEN):
                    hv = jnp.full((nl,), h, jnp.int32)
                    g = plsc.load_gather(slab_v, [hv, loc])
                    plsc.store_scatter(
                        xt_v, [hv, cols], jnp.where(m, g, jnp.float32(0.0)))

            def miss():
                for h in range(_HIDDEN):
                    hv = jnp.full((nl,), h, jnp.int32)
                    plsc.store_scatter(xt_v, [hv, cols], zeros)

            lax.cond(jnp.any(m), hit, miss)
            return carry

        lax.fori_loop(0, _BATCH // nl, chunk, 0, unroll=False)
        pltpu.sync_copy(xt_v, out_hbm.at[wid])

    return gather_k


def _matmul_body(xs_ref, projt_ref, out_ref, xt_vmem):
    i = pl.program_id(0)

    @pl.when(i == 0)
    def _():
        xt_vmem[...] = jnp.sum(xs_ref[...], axis=0)

    out_ref[...] = lax.dot_general(
        projt_ref[...],
        xt_vmem[...],
        dimension_numbers=(((0,), (0,)), ((), ())),
        preferred_element_type=jnp.float32,
    )


def _tc_matmul(xs, projt):
    nw = xs.shape[0]
    return pl.pallas_call(
        _matmul_body,
        grid=(pl.cdiv(_NUM_CHARS, _TM),),
        in_specs=[
            pl.BlockSpec((nw, _HIDDEN, _BATCH), lambda i: (0, 0, 0)),
            pl.BlockSpec((_HIDDEN, _TM), lambda i: (0, i)),
        ],
        out_specs=pl.BlockSpec((_TM, _BATCH), lambda i: (i, 0)),
        out_shape=jax.ShapeDtypeStruct((_NUM_CHARS, _BATCH), jnp.float32),
        scratch_shapes=[pltpu.VMEM((_HIDDEN, _BATCH), jnp.float32)],
    )(xs, projt)


def kernel(indices, embedding, projection_matrix):
    xs = _make_sc_gather()(indices.astype(jnp.int32), embedding.T)
    return _tc_matmul(xs, projection_matrix.T)
